# baseline (speedup 1.0000x reference)
import functools

import jax
import jax.numpy as jnp
from jax import lax
from jax.experimental import pallas as pl
from jax.experimental.pallas import tpu as pltpu
from jax.experimental.pallas import tpu_sc as plsc

B = 16384
E = 64
A = 18
CHUNK = 128


def _sc_info():
    try:
        info = plsc.get_sparse_core_info()
        return info.num_cores, info.num_subcores
    except Exception:
        return 2, 16


def _sc_gather(s0, s1, s2, embed):
    NC, NS = _sc_info()
    NW = NC * NS
    bpw = B // NW
    nch = bpw // CHUNK
    mesh = plsc.VectorSubcoreMesh(core_axis_name="c", subcore_axis_name="s")

    @functools.partial(
        pl.kernel,
        out_type=jax.ShapeDtypeStruct((B, E), jnp.float32),
        mesh=mesh,
        scratch_types=[
            pltpu.VMEM((bpw,), jnp.int32),
            pltpu.VMEM((bpw,), jnp.int32),
            pltpu.VMEM((bpw,), jnp.int32),
            pltpu.VMEM((nch, CHUNK), jnp.int32),
            pltpu.VMEM((bpw, E), jnp.float32),
            pltpu.SemaphoreType.DMA,
        ],
        compiler_params=pltpu.CompilerParams(use_tc_tiling_on_sc=False),
    )
    def gather_kernel(s0_hbm, s1_hbm, s2_hbm, embed_hbm, e_out,
                      s0_v, s1_v, s2_v, ids_v, rows_v, sem):
        wid = lax.axis_index("s") * NC + lax.axis_index("c")
        base = wid * bpw
        pltpu.sync_copy(s0_hbm.at[pl.ds(base, bpw)], s0_v)
        pltpu.sync_copy(s1_hbm.at[pl.ds(base, bpw)], s1_v)
        pltpu.sync_copy(s2_hbm.at[pl.ds(base, bpw)], s2_v)
        for g in range(bpw // 16):
            sl = pl.ds(g * 16, 16)
            ids = s0_v[sl] * 10000 + s1_v[sl] * 100 + s2_v[sl]
            ids_v[(g * 16) // CHUNK, pl.ds((g * 16) % CHUNK, 16)] = ids
        copies = [
            pltpu.async_copy(
                embed_hbm.at[ids_v.at[j]],
                rows_v.at[pl.ds(j * CHUNK, CHUNK)],
                sem,
            )
            for j in range(nch)
        ]
        for cpy in copies:
            cpy.wait()
        pltpu.sync_copy(rows_v, e_out.at[pl.ds(base, bpw)])

    return gather_kernel(s0, s1, s2, embed)


def _tc_matmul(e, W, b):
    blk = 2048

    def mm(e_ref, w_ref, b_ref, o_ref):
        o_ref[...] = (
            lax.dot_general(
                e_ref[...], w_ref[...],
                (((1,), (1,)), ((), ())),
                preferred_element_type=jnp.float32,
            )
            + b_ref[...]
        )

    return pl.pallas_call(
        mm,
        grid=(B // blk,),
        in_specs=[
            pl.BlockSpec((blk, E), lambda i: (i, 0)),
            pl.BlockSpec((A, E), lambda i: (0, 0)),
            pl.BlockSpec((1, A), lambda i: (0, 0)),
        ],
        out_specs=pl.BlockSpec((blk, A), lambda i: (i, 0)),
        out_shape=jax.ShapeDtypeStruct((B, A), jnp.float32),
    )(e, W, b.reshape(1, A))


def kernel(state, embed, W, b):
    e = _sc_gather(state[:, 0], state[:, 1], state[:, 2], embed)
    return _tc_matmul(e, W, b)
